# baseline (speedup 1.0000x reference)
import functools

import jax
import jax.numpy as jnp
from jax import lax
from jax.experimental import pallas as pl
from jax.experimental.pallas import tpu as pltpu
from jax.experimental.pallas import tpu_sc as plsc

D_MODEL = 128
MAX_LEN = 5000
NC, NS, LANES = 2, 16, 16
NW = NC * NS
SEQ = 512
JC = 128
ROWS_PER_W = SEQ // NW
ROW_GRP = 4


def _sc_body(x_hbm, rev_hbm, out_hbm, x_v, rev_v, o0, o1, o2, o3):
    outs = (o0, o1, o2, o3)
    wid = lax.axis_index("s") * NC + lax.axis_index("c")
    i_base = wid * ROWS_PER_W
    for jc in range(SEQ // JC):
        j0 = jc * JC
        pltpu.sync_copy(x_hbm.at[pl.ds(j0, JC), :], x_v)
        win0 = (SEQ - 1) - (i_base + ROWS_PER_W - 1) + j0
        pltpu.sync_copy(rev_hbm.at[pl.ds(win0, JC + ROWS_PER_W), :], rev_v)
        for rg in range(ROWS_PER_W // ROW_GRP):
            def body(jp, carry, rg=rg):
                for g in range(D_MODEL // LANES):
                    sl = pl.ds(g * LANES, LANES)
                    xv = x_v[jp, sl]
                    for r in range(ROW_GRP):
                        ii = rg * ROW_GRP + r
                        outs[r][jp, sl] = xv + rev_v[jp + (ROWS_PER_W - 1 - ii), sl]
                return carry
            lax.fori_loop(0, JC, body, 0)
            for r in range(ROW_GRP):
                i = i_base + rg * ROW_GRP + r
                pltpu.sync_copy(outs[r], out_hbm.at[i, pl.ds(j0, JC), :])


def kernel(inputs, rel_pos_encoding):
    _, seq_len, d = inputs.shape
    x = inputs[0]

    lo = MAX_LEN - (seq_len - 1)
    window = jax.lax.slice(rel_pos_encoding, (lo, 0), (lo + 2 * seq_len - 1, d))
    rev = jnp.flip(window, axis=0)
    rev = jnp.pad(rev, ((0, 1), (0, 0)))

    mesh = plsc.VectorSubcoreMesh(core_axis_name="c", subcore_axis_name="s")
    sc_call = pl.kernel(
        _sc_body,
        out_type=jax.ShapeDtypeStruct((seq_len, seq_len, d), inputs.dtype),
        mesh=mesh,
        scratch_types=[
            pltpu.VMEM((JC, d), jnp.float32),
            pltpu.VMEM((JC + ROWS_PER_W, d), jnp.float32),
        ] + [pltpu.VMEM((JC, d), jnp.float32) for _ in range(ROW_GRP)],
    )
    return sc_call(x, rev)


# speedup vs baseline: 2.0965x; 2.0965x over previous
import jax
import jax.numpy as jnp
from jax import lax
from jax.experimental import pallas as pl
from jax.experimental.pallas import tpu as pltpu
from jax.experimental.pallas import tpu_sc as plsc

D_MODEL = 128
MAX_LEN = 5000
NC, NS, LANES = 2, 16, 16
NW = NC * NS


def _sc_reverse_body(win_hbm, rev_hbm, buf, rbuf):
    rows = win_hbm.shape[0] // NW
    wid = lax.axis_index("s") * NC + lax.axis_index("c")
    r0 = wid * rows
    pltpu.sync_copy(win_hbm.at[pl.ds(r0, rows), :], buf)
    for t in range(rows):
        for g in range(D_MODEL // LANES):
            sl = pl.ds(g * LANES, LANES)
            rbuf[t, sl] = buf[rows - 1 - t, sl]
    out0 = win_hbm.shape[0] - rows - r0
    pltpu.sync_copy(rbuf, rev_hbm.at[pl.ds(out0, rows), :])


def _tc_add_body(x_ref, rev_ref, o_ref, *, block_i, seq_len):
    i0 = pl.program_id(0) * block_i
    for ii in range(block_i):
        start = (seq_len - 1) - i0 - ii
        o_ref[ii] = x_ref[:] + rev_ref[pl.ds(start, seq_len), :]


def kernel(inputs, rel_pos_encoding):
    _, seq_len, d = inputs.shape
    x = inputs[0]

    lo = MAX_LEN - seq_len
    window = jax.lax.slice(rel_pos_encoding, (lo, 0), (lo + 2 * seq_len, d))

    mesh = plsc.VectorSubcoreMesh(core_axis_name="c", subcore_axis_name="s")
    rows = 2 * seq_len // NW
    rev = pl.kernel(
        _sc_reverse_body,
        out_type=jax.ShapeDtypeStruct((2 * seq_len, d), jnp.float32),
        mesh=mesh,
        scratch_types=[
            pltpu.VMEM((rows, d), jnp.float32),
            pltpu.VMEM((rows, d), jnp.float32),
        ],
    )(window)

    block_i = 16
    grid = seq_len // block_i
    out = pl.pallas_call(
        lambda x_ref, rev_ref, o_ref: _tc_add_body(
            x_ref, rev_ref, o_ref, block_i=block_i, seq_len=seq_len
        ),
        grid=(grid,),
        in_specs=[
            pl.BlockSpec((seq_len, d), lambda g: (0, 0)),
            pl.BlockSpec((2 * seq_len, d), lambda g: (0, 0)),
        ],
        out_specs=pl.BlockSpec((block_i, seq_len, d), lambda g: (g, 0, 0)),
        out_shape=jax.ShapeDtypeStruct((seq_len, seq_len, d), inputs.dtype),
    )(x, rev)
    return out


# speedup vs baseline: 2.1098x; 1.0064x over previous
import jax
import jax.numpy as jnp
from jax import lax
from jax.experimental import pallas as pl
from jax.experimental.pallas import tpu as pltpu
from jax.experimental.pallas import tpu_sc as plsc

D_MODEL = 128
MAX_LEN = 5000
NC, NS, LANES = 2, 16, 16
NW = NC * NS


def _sc_reverse_body(table_hbm, rev_hbm, buf, rbuf, *, lo, n_cores, n_workers):
    n_rev = rev_hbm.shape[0]
    rows = n_rev // n_workers
    wid = lax.axis_index("s") * n_cores + lax.axis_index("c")
    r0 = wid * rows
    pltpu.sync_copy(table_hbm.at[pl.ds(lo + r0, rows), :], buf)
    for t in range(rows):
        for g in range(D_MODEL // LANES):
            sl = pl.ds(g * LANES, LANES)
            rbuf[t, sl] = buf[rows - 1 - t, sl]
    out0 = n_rev - rows - r0
    pltpu.sync_copy(rbuf, rev_hbm.at[pl.ds(out0, rows), :])


def _tc_add_body(x_ref, rev_ref, o_ref, *, block_i, seq_len):
    i0 = pl.program_id(0) * block_i
    for ii in range(block_i):
        start = (seq_len - 1) - i0 - ii
        o_ref[ii] = x_ref[:] + rev_ref[pl.ds(start, seq_len), :]


def kernel(inputs, rel_pos_encoding):
    _, seq_len, d = inputs.shape
    x = inputs[0]

    lo = MAX_LEN - seq_len

    mesh = plsc.VectorSubcoreMesh(
        core_axis_name="c", subcore_axis_name="s", num_cores=1
    )
    n_workers = 1 * NS
    rows = 2 * seq_len // n_workers
    rev = pl.kernel(
        lambda t, r, b0, b1: _sc_reverse_body(
            t, r, b0, b1, lo=lo, n_cores=1, n_workers=n_workers
        ),
        out_type=jax.ShapeDtypeStruct((2 * seq_len, d), jnp.float32),
        mesh=mesh,
        scratch_types=[
            pltpu.VMEM((rows, d), jnp.float32),
            pltpu.VMEM((rows, d), jnp.float32),
        ],
    )(rel_pos_encoding)

    block_i = 16
    grid = seq_len // block_i
    out = pl.pallas_call(
        lambda x_ref, rev_ref, o_ref: _tc_add_body(
            x_ref, rev_ref, o_ref, block_i=block_i, seq_len=seq_len
        ),
        grid=(grid,),
        in_specs=[
            pl.BlockSpec((seq_len, d), lambda g: (0, 0)),
            pl.BlockSpec((2 * seq_len, d), lambda g: (0, 0)),
        ],
        out_specs=pl.BlockSpec((block_i, seq_len, d), lambda g: (g, 0, 0)),
        out_shape=jax.ShapeDtypeStruct((seq_len, seq_len, d), inputs.dtype),
    )(x, rev)
    return out
